# baseline (speedup 1.0000x reference)
import jax
import jax.numpy as jnp
import numpy as np
from jax.experimental import pallas as pl
from jax.experimental.pallas import tpu as pltpu
from jax.experimental.pallas import tpu_sc as plsc
from jax import lax

T = 1000
TPAD = 1024
B = 128
R = 1176
C = 128
L = R * C
BB = 2

_K1 = np.uint32(42)
_K2 = np.uint32(0x1BD11BDA ^ 42)

_PC = (np.float32(1.2515922), np.float32(-0.23166679),
       np.float32(0.005711901), np.float32(0.0009422927),
       np.float32(2.9009212e-05))
_UMAX = np.float32(0.99999994)


def _table_body(beta_ref, t1_ref, t2_ref):
    la = jnp.log1p(-beta_ref[0, :])
    i = jax.lax.broadcasted_iota(jnp.int32, (TPAD, TPAD), 0)
    j = jax.lax.broadcasted_iota(jnp.int32, (TPAD, TPAD), 1)
    s = jnp.sum(jnp.where(j <= i, la[None, :], 0.0), axis=1, keepdims=True)
    cp = jnp.exp(s)
    t1_ref[...] = jnp.broadcast_to(jnp.sqrt(cp), (TPAD, 128))
    t2_ref[...] = jnp.broadcast_to(jnp.sqrt(1.0 - cp), (TPAD, 128))


_SC_W = 16
_SC_BPW = B // _SC_W


def _sc_gather_body(t1_hbm, t2_hbm, ts_hbm, o1_hbm, o2_hbm,
                    idx_v, r1_v, r2_v, sem1, sem2):
    wid = lax.axis_index("s") * 2 + lax.axis_index("c")

    @pl.when(wid < _SC_W)
    def _():
        base = wid * _SC_BPW
        pltpu.sync_copy(ts_hbm.at[pl.ds(base, _SC_BPW)], idx_v)
        c1 = pltpu.async_copy(t1_hbm.at[idx_v], r1_v, sem1)
        c2 = pltpu.async_copy(t2_hbm.at[idx_v], r2_v, sem2)
        c1.wait()
        c2.wait()
        pltpu.sync_copy(r1_v, o1_hbm.at[pl.ds(base, _SC_BPW)])
        pltpu.sync_copy(r2_v, o2_hbm.at[pl.ds(base, _SC_BPW)])


def _rotl(v, r):
    return (v << np.uint32(r)) | (v >> np.uint32(32 - r))


def _main_body(sa_ref, sb_ref, x_ref, out_ref, noise_ref):
    pid = pl.program_id(0)
    ir = jax.lax.broadcasted_iota(jnp.uint32, (R, C), 0)
    ic = jax.lax.broadcasted_iota(jnp.uint32, (R, C), 1)
    jb = ir * np.uint32(C) + ic + np.uint32(42)
    for bi in range(BB):
        b = pid * BB + bi
        base = (b * L).astype(jnp.uint32)
        x1 = jb + base
        x0 = x1
        x1 = _rotl(x1, 13) ^ x0
        for r in (15, 26, 6):
            x0 = x0 + x1
            x1 = _rotl(x1, r) ^ x0
        x0 = x0 + _K1
        x1 = x1 + np.uint32(_K2 + 1)
        for r in (17, 29, 16, 24):
            x0 = x0 + x1
            x1 = _rotl(x1, r) ^ x0
        x0 = x0 + _K2
        x1 = x1 + np.uint32(2)
        for r in (13, 15, 26, 6):
            x0 = x0 + x1
            x1 = _rotl(x1, r) ^ x0
        x1 = x1 + np.uint32(_K1 + 3)
        for r in (17, 29, 16, 24):
            x0 = x0 + x1
            x1 = _rotl(x1, r) ^ x0
        x0 = x0 + _K1
        x1 = x1 + np.uint32(_K2 + 4)
        for r in (13, 15, 26, 6):
            x0 = x0 + x1
            x1 = _rotl(x1, r) ^ x0
        x0 = x0 + _K2
        x1 = x1 + np.uint32((5 + 0x80000000) & 0xFFFFFFFF)
        sbits = jax.lax.bitcast_convert_type(x0 ^ x1, jnp.int32)

        u = sbits.astype(jnp.float32) * np.float32(2.0 ** -31)
        u = jnp.minimum(jnp.maximum(u, -_UMAX), _UMAX)
        y = jnp.log2(1.0 - u * u)
        p = (((_PC[4] * y + _PC[3]) * y + _PC[2]) * y + _PC[1]) * y + _PC[0]
        z = u * p

        noise_ref[bi] = z
        sa = sa_ref[b, 0]
        sb = sb_ref[b, 0]
        out_ref[bi] = sa * x_ref[bi] + sb * z


@jax.jit
def kernel(x, timestep, beta_schedule):
    beta = jnp.pad(beta_schedule, (0, TPAD - T)).reshape(1, TPAD)
    t1, t2 = pl.pallas_call(
        _table_body,
        in_specs=[pl.BlockSpec((1, TPAD), lambda: (0, 0))],
        out_specs=[
            pl.BlockSpec((TPAD, 128), lambda: (0, 0)),
            pl.BlockSpec((TPAD, 128), lambda: (0, 0)),
        ],
        out_shape=[
            jax.ShapeDtypeStruct((TPAD, 128), jnp.float32),
            jax.ShapeDtypeStruct((TPAD, 128), jnp.float32),
        ],
    )(beta)

    mesh = plsc.VectorSubcoreMesh(core_axis_name="c", subcore_axis_name="s")
    sc_gather = pl.kernel(
        _sc_gather_body,
        mesh=mesh,
        out_type=[
            jax.ShapeDtypeStruct((B, 128), jnp.float32),
            jax.ShapeDtypeStruct((B, 128), jnp.float32),
        ],
        scratch_types=[
            pltpu.VMEM((_SC_BPW,), jnp.int32),
            pltpu.VMEM((_SC_BPW, 128), jnp.float32),
            pltpu.VMEM((_SC_BPW, 128), jnp.float32),
            pltpu.SemaphoreType.DMA,
            pltpu.SemaphoreType.DMA,
        ],
    )
    g1, g2 = sc_gather(t1, t2, timestep)
    sa = g1[:, :1]
    sb = g2[:, :1]

    x3 = x.reshape(B, R, C)
    out, noise = pl.pallas_call(
        _main_body,
        grid=(B // BB,),
        in_specs=[
            pl.BlockSpec(memory_space=pltpu.SMEM),
            pl.BlockSpec(memory_space=pltpu.SMEM),
            pl.BlockSpec((BB, R, C), lambda i: (i, 0, 0)),
        ],
        out_specs=[
            pl.BlockSpec((BB, R, C), lambda i: (i, 0, 0)),
            pl.BlockSpec((BB, R, C), lambda i: (i, 0, 0)),
        ],
        out_shape=[
            jax.ShapeDtypeStruct((B, R, C), x.dtype),
            jax.ShapeDtypeStruct((B, R, C), x.dtype),
        ],
    )(sa, sb, x3)
    return out.reshape(x.shape), noise.reshape(x.shape)


# speedup vs baseline: 1.0078x; 1.0078x over previous
import jax
import jax.numpy as jnp
import numpy as np
from jax.experimental import pallas as pl
from jax.experimental.pallas import tpu as pltpu
from jax.experimental.pallas import tpu_sc as plsc
from jax import lax

T = 1000
TPAD = 1024
B = 128
R = 1176
C = 128
L = R * C
BB = 2

_K1 = np.uint32(42)
_K2 = np.uint32(0x1BD11BDA ^ 42)

_PC = (np.float32(1.2515922), np.float32(-0.23166679),
       np.float32(0.005711901), np.float32(0.0009422927),
       np.float32(2.9009212e-05))
_UMAX = np.float32(0.99999994)


def _table_body(beta_ref, t1_ref, t2_ref):
    la = jnp.log1p(-beta_ref[0, :])
    i = jax.lax.broadcasted_iota(jnp.int32, (TPAD, TPAD), 0)
    j = jax.lax.broadcasted_iota(jnp.int32, (TPAD, TPAD), 1)
    s = jnp.sum(jnp.where(j <= i, la[None, :], 0.0), axis=1, keepdims=True)
    cp = jnp.exp(s)
    t1_ref[...] = jnp.broadcast_to(jnp.sqrt(cp), (TPAD, 128))
    t2_ref[...] = jnp.broadcast_to(jnp.sqrt(1.0 - cp), (TPAD, 128))


_SC_W = 16
_SC_BPW = B // _SC_W


def _sc_gather_body(t1_hbm, t2_hbm, ts_hbm, o1_hbm, o2_hbm,
                    idx_v, r1_v, r2_v, sem1, sem2):
    wid = lax.axis_index("s") * 2 + lax.axis_index("c")

    @pl.when(wid < _SC_W)
    def _():
        base = wid * _SC_BPW
        pltpu.sync_copy(ts_hbm.at[pl.ds(base, _SC_BPW)], idx_v)
        c1 = pltpu.async_copy(t1_hbm.at[idx_v], r1_v, sem1)
        c2 = pltpu.async_copy(t2_hbm.at[idx_v], r2_v, sem2)
        c1.wait()
        c2.wait()
        pltpu.sync_copy(r1_v, o1_hbm.at[pl.ds(base, _SC_BPW)])
        pltpu.sync_copy(r2_v, o2_hbm.at[pl.ds(base, _SC_BPW)])


def _rotl(v, r):
    return (v << np.uint32(r)) | (v >> np.uint32(32 - r))


def _main_body(sa_ref, sb_ref, jb_ref, x_ref, out_ref, noise_ref):
    pid = pl.program_id(0)
    for bi in range(BB):
        b = pid * BB + bi
        base = (b * L).astype(jnp.uint32)
        x1 = jb_ref[0] + base
        x0 = x1
        x1 = _rotl(x1, 13) ^ x0
        for r in (15, 26, 6):
            x0 = x0 + x1
            x1 = _rotl(x1, r) ^ x0
        x0 = x0 + _K1
        x1 = x1 + np.uint32(_K2 + 1)
        for r in (17, 29, 16, 24):
            x0 = x0 + x1
            x1 = _rotl(x1, r) ^ x0
        x0 = x0 + _K2
        x1 = x1 + np.uint32(2)
        for r in (13, 15, 26, 6):
            x0 = x0 + x1
            x1 = _rotl(x1, r) ^ x0
        x1 = x1 + np.uint32(_K1 + 3)
        for r in (17, 29, 16, 24):
            x0 = x0 + x1
            x1 = _rotl(x1, r) ^ x0
        x0 = x0 + _K1
        x1 = x1 + np.uint32(_K2 + 4)
        for r in (13, 15, 26, 6):
            x0 = x0 + x1
            x1 = _rotl(x1, r) ^ x0
        x0 = x0 + _K2
        x1 = x1 + np.uint32((5 + 0x80000000) & 0xFFFFFFFF)
        sbits = jax.lax.bitcast_convert_type(x0 ^ x1, jnp.int32)

        u = sbits.astype(jnp.float32) * np.float32(2.0 ** -31)
        u = jnp.minimum(jnp.maximum(u, -_UMAX), _UMAX)
        y = jnp.log2(1.0 - u * u)
        p = (((_PC[4] * y + _PC[3]) * y + _PC[2]) * y + _PC[1]) * y + _PC[0]
        z = u * p

        noise_ref[bi] = z
        sa = sa_ref[b, 0]
        sb = sb_ref[b, 0]
        out_ref[bi] = sa * x_ref[bi] + sb * z


@jax.jit
def kernel(x, timestep, beta_schedule):
    beta = jnp.pad(beta_schedule, (0, TPAD - T)).reshape(1, TPAD)
    t1, t2 = pl.pallas_call(
        _table_body,
        in_specs=[pl.BlockSpec((1, TPAD), lambda: (0, 0))],
        out_specs=[
            pl.BlockSpec((TPAD, 128), lambda: (0, 0)),
            pl.BlockSpec((TPAD, 128), lambda: (0, 0)),
        ],
        out_shape=[
            jax.ShapeDtypeStruct((TPAD, 128), jnp.float32),
            jax.ShapeDtypeStruct((TPAD, 128), jnp.float32),
        ],
    )(beta)

    mesh = plsc.VectorSubcoreMesh(core_axis_name="c", subcore_axis_name="s")
    sc_gather = pl.kernel(
        _sc_gather_body,
        mesh=mesh,
        out_type=[
            jax.ShapeDtypeStruct((B, 128), jnp.float32),
            jax.ShapeDtypeStruct((B, 128), jnp.float32),
        ],
        scratch_types=[
            pltpu.VMEM((_SC_BPW,), jnp.int32),
            pltpu.VMEM((_SC_BPW, 128), jnp.float32),
            pltpu.VMEM((_SC_BPW, 128), jnp.float32),
            pltpu.SemaphoreType.DMA,
            pltpu.SemaphoreType.DMA,
        ],
    )
    g1, g2 = sc_gather(t1, t2, timestep)
    sa = g1[:, :1]
    sb = g2[:, :1]

    jb = (jnp.arange(L, dtype=jnp.uint32) + jnp.uint32(42)).reshape(1, R, C)
    x3 = x.reshape(B, R, C)
    out, noise = pl.pallas_call(
        _main_body,
        grid=(B // BB,),
        in_specs=[
            pl.BlockSpec(memory_space=pltpu.SMEM),
            pl.BlockSpec(memory_space=pltpu.SMEM),
            pl.BlockSpec((1, R, C), lambda i: (0, 0, 0)),
            pl.BlockSpec((BB, R, C), lambda i: (i, 0, 0)),
        ],
        out_specs=[
            pl.BlockSpec((BB, R, C), lambda i: (i, 0, 0)),
            pl.BlockSpec((BB, R, C), lambda i: (i, 0, 0)),
        ],
        out_shape=[
            jax.ShapeDtypeStruct((B, R, C), x.dtype),
            jax.ShapeDtypeStruct((B, R, C), x.dtype),
        ],
    )(sa, sb, jb, x3)
    return out.reshape(x.shape), noise.reshape(x.shape)
